# Initial kernel scaffold; baseline (speedup 1.0000x reference)
import functools

import jax
import jax.numpy as jnp
from jax import lax
from jax.experimental import pallas as pl
from jax.experimental.pallas import tpu as pltpu
from jax.experimental.pallas import tpu_sc as plsc

LANES = 16


def _build_sc_call(B, F, V, D, L, SD, NC, NS):
    NW = NC * NS
    BPW = B // NW
    BC = 16
    NCH = BPW // BC
    mesh = plsc.VectorSubcoreMesh(core_axis_name="c", subcore_axis_name="s")

    @functools.partial(
        pl.kernel,
        out_type=jax.ShapeDtypeStruct((B, F * D + SD), jnp.float32),
        mesh=mesh,
        scratch_types=[
            pltpu.VMEM((BC * F,), jnp.int32),
            pltpu.VMEM((BC * F, D), jnp.float32),
            pltpu.VMEM((BC * L,), jnp.int32),
            pltpu.VMEM((BC * L, SD), jnp.float32),
            pltpu.VMEM((BC, SD), jnp.float32),
            pltpu.SemaphoreType.DMA,
        ],
    )
    def run(cat_idx_h, seq_idx_h, ctab_h, stab_h, out_h,
            ci_v, cr_v, si_v, sr_v, so_v, sem):
        w = lax.axis_index("s") * NC + lax.axis_index("c")
        base = w * BPW

        def chunk(i, carry):
            r0 = base + i * BC
            pltpu.sync_copy(cat_idx_h.at[pl.ds(r0 * F, BC * F)], ci_v)
            pltpu.async_copy(ctab_h.at[ci_v], cr_v, sem).wait()
            pltpu.sync_copy(seq_idx_h.at[pl.ds(r0 * L, BC * L)], si_v)
            pltpu.async_copy(stab_h.at[si_v], sr_v, sem).wait()

            def row(r, carry2):
                def acc_step(l, accs):
                    return tuple(
                        accs[j] + sr_v[r * L + l, pl.ds(j * LANES, LANES)]
                        for j in range(SD // LANES)
                    )
                zeros = tuple(jnp.zeros((LANES,), jnp.float32)
                              for _ in range(SD // LANES))
                accs = lax.fori_loop(0, L, acc_step, zeros)
                for j in range(SD // LANES):
                    so_v[r, pl.ds(j * LANES, LANES)] = accs[j] * (1.0 / L)
                return carry2

            lax.fori_loop(0, BC, row, 0)
            pltpu.sync_copy(cr_v.reshape(BC, F * D),
                            out_h.at[pl.ds(r0, BC), pl.ds(0, F * D)])
            pltpu.sync_copy(so_v, out_h.at[pl.ds(r0, BC), pl.ds(F * D, SD)])
            return carry

        lax.fori_loop(0, NCH, chunk, 0)

    return run


def kernel(cat_idx, seq_ids, cat_tables, seq_table):
    B, F = cat_idx.shape
    L = seq_ids.shape[1]
    _, V, D = cat_tables.shape
    SD = seq_table.shape[1]
    info = plsc.get_sparse_core_info()
    NC, NS = info.num_cores, info.num_subcores

    flat_tables = cat_tables.reshape(F * V, D)
    cat_flat = (cat_idx.astype(jnp.int32)
                + (jnp.arange(F, dtype=jnp.int32) * V)[None, :]).reshape(B * F)
    seq_flat = seq_ids.astype(jnp.int32).reshape(B * L)

    run = _build_sc_call(B, F, V, D, L, SD, NC, NS)
    return run(cat_flat, seq_flat, flat_tables, seq_table)


# speedup vs baseline: 3.2011x; 3.2011x over previous
import functools

import jax
import jax.numpy as jnp
from jax import lax
from jax.experimental import pallas as pl
from jax.experimental.pallas import tpu as pltpu
from jax.experimental.pallas import tpu_sc as plsc

LANES = 16


def _build_sc_call(B, F, V, D, L, SD, NC, NS):
    NW = NC * NS
    BPW = B // NW
    BC = 8
    NCH = BPW // BC
    OW = F * D + SD
    mesh = plsc.VectorSubcoreMesh(core_axis_name="c", subcore_axis_name="s")

    @functools.partial(
        pl.kernel,
        out_type=jax.ShapeDtypeStruct((B, OW), jnp.float32),
        mesh=mesh,
        compiler_params=pltpu.CompilerParams(use_tc_tiling_on_sc=False),
        scratch_types=[
            pltpu.VMEM((BC * F,), jnp.int32),
            pltpu.VMEM((BC * F, D), jnp.float32),
            pltpu.VMEM((BC * L,), jnp.int32),
            pltpu.VMEM((BC * L, SD), jnp.float32),
            pltpu.VMEM((BC, OW), jnp.float32),
            pltpu.SemaphoreType.DMA,
        ],
    )
    def run(cat_idx_h, seq_idx_h, ctab_h, stab_h, out_h,
            ci_v, cr_v, si_v, sr_v, row_v, sem):
        w = lax.axis_index("s") * NC + lax.axis_index("c")
        base = w * BPW

        def chunk(i, carry):
            r0 = base + i * BC
            pltpu.sync_copy(cat_idx_h.at[pl.ds(r0 * F, BC * F)], ci_v)
            pltpu.async_copy(ctab_h.at[ci_v], cr_v, sem).wait()
            pltpu.sync_copy(seq_idx_h.at[pl.ds(r0 * L, BC * L)], si_v)
            pltpu.async_copy(stab_h.at[si_v], sr_v, sem).wait()

            def row(r, carry2):
                for k in range(F * D // LANES):
                    f, j = divmod(k, D // LANES)
                    row_v[r, pl.ds(k * LANES, LANES)] = (
                        cr_v[r * F + f, pl.ds(j * LANES, LANES)])

                def acc_step(l, accs):
                    return tuple(
                        accs[j] + sr_v[r * L + l, pl.ds(j * LANES, LANES)]
                        for j in range(SD // LANES)
                    )
                zeros = tuple(jnp.zeros((LANES,), jnp.float32)
                              for _ in range(SD // LANES))
                accs = lax.fori_loop(0, L, acc_step, zeros)
                for j in range(SD // LANES):
                    row_v[r, pl.ds(F * D + j * LANES, LANES)] = (
                        accs[j] * (1.0 / L))
                return carry2

            lax.fori_loop(0, BC, row, 0)
            pltpu.sync_copy(row_v, out_h.at[pl.ds(r0, BC)])
            return carry

        lax.fori_loop(0, NCH, chunk, 0)

    return run


def kernel(cat_idx, seq_ids, cat_tables, seq_table):
    B, F = cat_idx.shape
    L = seq_ids.shape[1]
    _, V, D = cat_tables.shape
    SD = seq_table.shape[1]
    info = plsc.get_sparse_core_info()
    NC, NS = info.num_cores, info.num_subcores

    flat_tables = cat_tables.reshape(F * V, D)
    cat_flat = (cat_idx.astype(jnp.int32)
                + (jnp.arange(F, dtype=jnp.int32) * V)[None, :]).reshape(B * F)
    seq_flat = seq_ids.astype(jnp.int32).reshape(B * L)

    run = _build_sc_call(B, F, V, D, L, SD, NC, NS)
    return run(cat_flat, seq_flat, flat_tables, seq_table)


# speedup vs baseline: 3.3150x; 1.0356x over previous
import functools

import jax
import jax.numpy as jnp
from jax import lax
from jax.experimental import pallas as pl
from jax.experimental.pallas import tpu as pltpu
from jax.experimental.pallas import tpu_sc as plsc

LANES = 16


def _build_sc_call(B, F, V, D, L, SD, NC, NS):
    NW = NC * NS
    BPW = B // NW
    BC = 8
    NCH = BPW // BC
    OW = F * D + SD
    NT = OW // 128
    mesh = plsc.VectorSubcoreMesh(core_axis_name="c", subcore_axis_name="s")

    @functools.partial(
        pl.kernel,
        out_type=jax.ShapeDtypeStruct((B // BC, NT, BC, 128), jnp.float32),
        mesh=mesh,
        compiler_params=pltpu.CompilerParams(use_tc_tiling_on_sc=False),
        scratch_types=[
            pltpu.VMEM((BC * F,), jnp.int32),
            pltpu.VMEM((BC * F, D), jnp.float32),
            pltpu.VMEM((BC * L,), jnp.int32),
            pltpu.VMEM((BC * L, SD), jnp.float32),
            pltpu.VMEM((NT, BC, 128), jnp.float32),
            pltpu.SemaphoreType.DMA,
        ],
    )
    def run(cat_idx_h, seq_idx_h, ctab_h, stab_h, out_h,
            ci_v, cr_v, si_v, sr_v, row_v, sem):
        w = lax.axis_index("s") * NC + lax.axis_index("c")
        base = w * BPW

        def chunk(i, carry):
            r0 = base + i * BC
            pltpu.sync_copy(cat_idx_h.at[pl.ds(r0 * F, BC * F)], ci_v)
            pltpu.async_copy(ctab_h.at[ci_v], cr_v, sem).wait()
            pltpu.sync_copy(seq_idx_h.at[pl.ds(r0 * L, BC * L)], si_v)
            pltpu.async_copy(stab_h.at[si_v], sr_v, sem).wait()

            def row(r, carry2):
                for k in range(F * D // LANES):
                    f, j = divmod(k, D // LANES)
                    col = k * LANES
                    row_v[col // 128, r, pl.ds(col % 128, LANES)] = (
                        cr_v[r * F + f, pl.ds(j * LANES, LANES)])

                def acc_step(l, accs):
                    return tuple(
                        accs[j] + sr_v[r * L + l, pl.ds(j * LANES, LANES)]
                        for j in range(SD // LANES)
                    )
                zeros = tuple(jnp.zeros((LANES,), jnp.float32)
                              for _ in range(SD // LANES))
                accs = lax.fori_loop(0, L, acc_step, zeros)
                for j in range(SD // LANES):
                    col = F * D + j * LANES
                    row_v[col // 128, r, pl.ds(col % 128, LANES)] = (
                        accs[j] * (1.0 / L))
                return carry2

            lax.fori_loop(0, BC, row, 0)
            pltpu.sync_copy(row_v, out_h.at[base // BC + i])
            return carry

        lax.fori_loop(0, NCH, chunk, 0)

    return run


def kernel(cat_idx, seq_ids, cat_tables, seq_table):
    B, F = cat_idx.shape
    L = seq_ids.shape[1]
    _, V, D = cat_tables.shape
    SD = seq_table.shape[1]
    info = plsc.get_sparse_core_info()
    NC, NS = info.num_cores, info.num_subcores

    flat_tables = cat_tables.reshape(F * V, D)
    cat_flat = (cat_idx.astype(jnp.int32)
                + (jnp.arange(F, dtype=jnp.int32) * V)[None, :]).reshape(B * F)
    seq_flat = seq_ids.astype(jnp.int32).reshape(B * L)

    run = _build_sc_call(B, F, V, D, L, SD, NC, NS)
    out4 = run(cat_flat, seq_flat, flat_tables, seq_table)
    return out4.transpose(0, 2, 1, 3).reshape(B, F * D + SD)


# speedup vs baseline: 3.9075x; 1.1787x over previous
import functools

import jax
import jax.numpy as jnp
from jax import lax
from jax.experimental import pallas as pl
from jax.experimental.pallas import tpu as pltpu
from jax.experimental.pallas import tpu_sc as plsc

LANES = 16


def _build_sc_call(B, F, V, D, L, SD, NC, NS):
    NW = NC * NS
    BPW = B // NW
    BC = 8
    NCH = BPW // BC
    OW = F * D + SD
    NT = OW // 128
    mesh = plsc.VectorSubcoreMesh(core_axis_name="c", subcore_axis_name="s")

    @functools.partial(
        pl.kernel,
        out_type=jax.ShapeDtypeStruct((B // BC, NT, BC, 128), jnp.float32),
        mesh=mesh,
        compiler_params=pltpu.CompilerParams(use_tc_tiling_on_sc=False),
        scratch_types=[
            pltpu.VMEM((BPW * F,), jnp.int32),
            pltpu.VMEM((BPW * L,), jnp.int32),
            pltpu.VMEM((BC * F, D), jnp.float32),
            pltpu.VMEM((BC * F, D), jnp.float32),
            pltpu.VMEM((BC * L, SD), jnp.float32),
            pltpu.VMEM((BC * L, SD), jnp.float32),
            pltpu.VMEM((NT, BC, 128), jnp.float32),
            pltpu.SemaphoreType.DMA,
            pltpu.SemaphoreType.DMA,
            pltpu.SemaphoreType.DMA,
            pltpu.SemaphoreType.DMA,
        ],
    )
    def run(cat_idx_h, seq_idx_h, ctab_h, stab_h, out_h,
            gi_v, si_v, cr0, cr1, sr0, sr1, row_v,
            semc0, semc1, sems0, sems1):
        w = lax.axis_index("s") * NC + lax.axis_index("c")
        base = w * BPW
        pltpu.sync_copy(cat_idx_h.at[pl.ds(base * F, BPW * F)], gi_v)
        pltpu.sync_copy(seq_idx_h.at[pl.ds(base * L, BPW * L)], si_v)

        crs = (cr0, cr1)
        srs = (sr0, sr1)
        semcs = (semc0, semc1)
        semss = (sems0, sems1)

        def start_gathers(i, p):
            pltpu.async_copy(
                ctab_h.at[gi_v.at[pl.ds(i * BC * F, BC * F)]],
                crs[p], semcs[p])
            pltpu.async_copy(
                stab_h.at[si_v.at[pl.ds(i * BC * L, BC * L)]],
                srs[p], semss[p])

        def wait_gathers(i, p):
            pltpu.make_async_copy(
                ctab_h.at[gi_v.at[pl.ds(i * BC * F, BC * F)]],
                crs[p], semcs[p]).wait()
            pltpu.make_async_copy(
                stab_h.at[si_v.at[pl.ds(i * BC * L, BC * L)]],
                srs[p], semss[p]).wait()

        def compute_and_store(i, p):
            cr_v, sr_v = crs[p], srs[p]

            def row(r, carry2):
                for f in range(F):
                    pf = r * F + f
                    for j in range(D // LANES):
                        col = f * D + j * LANES
                        row_v[col // 128, r, pl.ds(col % 128, LANES)] = (
                            cr_v[pf, pl.ds(j * LANES, LANES)])

                def acc_step(l, accs):
                    q = r * L + 2 * l
                    partial = tuple(
                        accs[j] + sr_v[q, pl.ds(j * LANES, LANES)]
                        for j in range(SD // LANES)
                    )
                    return tuple(
                        partial[j] + sr_v[q + 1, pl.ds(j * LANES, LANES)]
                        for j in range(SD // LANES)
                    )
                zeros = tuple(jnp.zeros((LANES,), jnp.float32)
                              for _ in range(SD // LANES))
                accs = lax.fori_loop(0, L // 2, acc_step, zeros)
                for j in range(SD // LANES):
                    col = F * D + j * LANES
                    row_v[col // 128, r, pl.ds(col % 128, LANES)] = (
                        accs[j] * (1.0 / L))
                return carry2

            lax.fori_loop(0, BC, row, 0)
            pltpu.sync_copy(row_v, out_h.at[base // BC + i])

        start_gathers(0, 0)

        def body(j2, carry):
            i = 2 * j2
            start_gathers(i + 1, 1)
            wait_gathers(i, 0)
            compute_and_store(i, 0)

            @pl.when(j2 < NCH // 2 - 1)
            def _():
                start_gathers(i + 2, 0)

            wait_gathers(i + 1, 1)
            compute_and_store(i + 1, 1)
            return carry

        lax.fori_loop(0, NCH // 2, body, 0)

    return run


def kernel(cat_idx, seq_ids, cat_tables, seq_table):
    B, F = cat_idx.shape
    L = seq_ids.shape[1]
    _, V, D = cat_tables.shape
    SD = seq_table.shape[1]
    info = plsc.get_sparse_core_info()
    NC, NS = info.num_cores, info.num_subcores

    flat_tables = cat_tables.reshape(F * V, D)
    cat_flat = (cat_idx.astype(jnp.int32)
                + (jnp.arange(F, dtype=jnp.int32) * V)[None, :]).reshape(B * F)
    seq_flat = seq_ids.astype(jnp.int32).reshape(B * L)

    run = _build_sc_call(B, F, V, D, L, SD, NC, NS)
    out4 = run(cat_flat, seq_flat, flat_tables, seq_table)
    return out4.transpose(0, 2, 1, 3).reshape(B, F * D + SD)


# speedup vs baseline: 4.0038x; 1.0247x over previous
import functools

import jax
import jax.numpy as jnp
from jax import lax
from jax.experimental import pallas as pl
from jax.experimental.pallas import tpu as pltpu
from jax.experimental.pallas import tpu_sc as plsc

LANES = 16


def _build_seq_call(B, L, SD, NC, NS):
    NW = NC * NS
    BPW = B // NW
    BC = 8
    NCH = BPW // BC
    mesh = plsc.VectorSubcoreMesh(core_axis_name="c", subcore_axis_name="s")

    @functools.partial(
        pl.kernel,
        out_type=jax.ShapeDtypeStruct((B // 2, 2 * SD), jnp.float32),
        mesh=mesh,
        compiler_params=pltpu.CompilerParams(use_tc_tiling_on_sc=False),
        scratch_types=[
            pltpu.VMEM((BPW * L,), jnp.int32),
            pltpu.VMEM((BC * L, SD), jnp.float32),
            pltpu.VMEM((BC * L, SD), jnp.float32),
            pltpu.VMEM((BPW // 2, 2 * SD), jnp.float32),
            pltpu.SemaphoreType.DMA,
            pltpu.SemaphoreType.DMA,
        ],
    )
    def run(seq_idx_h, stab_h, mn_h, si_v, sr0, sr1, mn_v, sem0, sem1):
        w = lax.axis_index("s") * NC + lax.axis_index("c")
        base = w * BPW
        pltpu.sync_copy(seq_idx_h.at[pl.ds(base * L, BPW * L)], si_v)
        srs = (sr0, sr1)
        sems = (sem0, sem1)

        def start_g(i, p):
            pltpu.async_copy(
                stab_h.at[si_v.at[pl.ds(i * BC * L, BC * L)]],
                srs[p], sems[p])

        def wait_g(i, p):
            pltpu.make_async_copy(
                stab_h.at[si_v.at[pl.ds(i * BC * L, BC * L)]],
                srs[p], sems[p]).wait()

        def accumulate(i, p):
            sr_v = srs[p]

            def row(r, carry2):
                def acc_step(l, accs):
                    q = r * L + 2 * l
                    partial = tuple(
                        accs[j] + sr_v[q, pl.ds(j * LANES, LANES)]
                        for j in range(SD // LANES)
                    )
                    return tuple(
                        partial[j] + sr_v[q + 1, pl.ds(j * LANES, LANES)]
                        for j in range(SD // LANES)
                    )
                zeros = tuple(jnp.zeros((LANES,), jnp.float32)
                              for _ in range(SD // LANES))
                accs = lax.fori_loop(0, L // 2, acc_step, zeros)
                li = i * BC + r
                half = (li % 2) * SD
                for j in range(SD // LANES):
                    mn_v[li // 2, pl.ds(half + j * LANES, LANES)] = (
                        accs[j] * (1.0 / L))
                return carry2

            lax.fori_loop(0, BC, row, 0)

        start_g(0, 0)

        def body(j2, carry):
            i = 2 * j2
            start_g(i + 1, 1)
            wait_g(i, 0)
            accumulate(i, 0)

            @pl.when(j2 < NCH // 2 - 1)
            def _():
                start_g(i + 2, 0)

            wait_g(i + 1, 1)
            accumulate(i + 1, 1)
            return carry

        lax.fori_loop(0, NCH // 2, body, 0)
        pltpu.sync_copy(mn_v, mn_h.at[pl.ds(base // 2, BPW // 2)])

    return run


def _build_cat_call(B, F, V, D, SD, NC, NS):
    NW = NC * NS
    BPW = B // NW
    BC = 8
    NCH = BPW // BC
    OW = F * D + SD
    NT = OW // 128
    mesh = plsc.VectorSubcoreMesh(core_axis_name="c", subcore_axis_name="s")

    @functools.partial(
        pl.kernel,
        out_type=jax.ShapeDtypeStruct((B // BC, NT, BC, 128), jnp.float32),
        mesh=mesh,
        compiler_params=pltpu.CompilerParams(use_tc_tiling_on_sc=False),
        scratch_types=[
            pltpu.VMEM((BPW * F,), jnp.int32),
            pltpu.VMEM((BPW // 2, 2 * SD), jnp.float32),
            pltpu.VMEM((BC * F, D), jnp.float32),
            pltpu.VMEM((BC * F, D), jnp.float32),
            pltpu.VMEM((NT, BC, 128), jnp.float32),
            pltpu.SemaphoreType.DMA,
            pltpu.SemaphoreType.DMA,
        ],
    )
    def run(cat_idx_h, ctab_h, mn_h, out_h,
            gi_v, mn_v, cr0, cr1, row_v, sem0, sem1):
        w = lax.axis_index("s") * NC + lax.axis_index("c")
        base = w * BPW
        pltpu.sync_copy(cat_idx_h.at[pl.ds(base * F, BPW * F)], gi_v)
        pltpu.sync_copy(mn_h.at[pl.ds(base // 2, BPW // 2)], mn_v)
        crs = (cr0, cr1)
        sems = (sem0, sem1)

        def start_g(i, p):
            pltpu.async_copy(
                ctab_h.at[gi_v.at[pl.ds(i * BC * F, BC * F)]],
                crs[p], sems[p])

        def wait_g(i, p):
            pltpu.make_async_copy(
                ctab_h.at[gi_v.at[pl.ds(i * BC * F, BC * F)]],
                crs[p], sems[p]).wait()

        def compute_and_store(i, p):
            cr_v = crs[p]

            def row(r, carry2):
                for f in range(F):
                    pf = r * F + f
                    for j in range(D // LANES):
                        col = f * D + j * LANES
                        row_v[col // 128, r, pl.ds(col % 128, LANES)] = (
                            cr_v[pf, pl.ds(j * LANES, LANES)])
                li = i * BC + r
                half = (li % 2) * SD
                for j in range(SD // LANES):
                    col = F * D + j * LANES
                    row_v[col // 128, r, pl.ds(col % 128, LANES)] = (
                        mn_v[li // 2, pl.ds(half + j * LANES, LANES)])
                return carry2

            lax.fori_loop(0, BC, row, 0)
            pltpu.sync_copy(row_v, out_h.at[base // BC + i])

        start_g(0, 0)

        def body(j2, carry):
            i = 2 * j2
            start_g(i + 1, 1)
            wait_g(i, 0)
            compute_and_store(i, 0)

            @pl.when(j2 < NCH // 2 - 1)
            def _():
                start_g(i + 2, 0)

            wait_g(i + 1, 1)
            compute_and_store(i + 1, 1)
            return carry

        lax.fori_loop(0, NCH // 2, body, 0)

    return run


def kernel(cat_idx, seq_ids, cat_tables, seq_table):
    B, F = cat_idx.shape
    L = seq_ids.shape[1]
    _, V, D = cat_tables.shape
    SD = seq_table.shape[1]
    info = plsc.get_sparse_core_info()
    NC, NS = info.num_cores, info.num_subcores

    flat_tables = cat_tables.reshape(F * V, D)
    cat_flat = (cat_idx.astype(jnp.int32)
                + (jnp.arange(F, dtype=jnp.int32) * V)[None, :]).reshape(B * F)
    seq_flat = seq_ids.astype(jnp.int32).reshape(B * L)

    seq_run = _build_seq_call(B, L, SD, NC, NS)
    means = seq_run(seq_flat, seq_table)
    cat_run = _build_cat_call(B, F, V, D, SD, NC, NS)
    out4 = cat_run(cat_flat, flat_tables, means)
    return out4.transpose(0, 2, 1, 3).reshape(B, F * D + SD)
